# Initial kernel scaffold; baseline (speedup 1.0000x reference)
import jax
import jax.numpy as jnp
from jax.experimental import pallas as pl


def kernel(z, codebook):
    raise NotImplementedError("write your pallas kernel here")


# speedup vs baseline: 1.3667x; 1.3667x over previous
import jax
import jax.numpy as jnp
from jax.experimental import pallas as pl

K = 8192
C = 32
T = 1024
KCH = 1024
NKCH = K // KCH


def _vq_kernel(z_ref, cb_ref, codes_ref, quant_ref, loss_ref):
    zb = z_ref[0]
    z2 = jnp.sum(zb * zb, axis=0, keepdims=True)
    best = jnp.full((1, T), jnp.inf, dtype=jnp.float32)
    bestidx = jnp.zeros((1, T), dtype=jnp.int32)
    for kc in range(NKCH):
        cb_chunk = cb_ref[kc * KCH:(kc + 1) * KCH, :]
        cn2 = jnp.sum(cb_chunk * cb_chunk, axis=1, keepdims=True)
        s = jax.lax.dot_general(cb_chunk, zb, (((1,), (0,)), ((), ())),
                                preferred_element_type=jnp.float32)
        d = (z2 + cn2) - 2.0 * s
        cmin = jnp.min(d, axis=0, keepdims=True)
        kio = jax.lax.broadcasted_iota(jnp.int32, (KCH, T), 0) + kc * KCH
        carg = jnp.min(jnp.where(d == cmin, kio, K), axis=0, keepdims=True)
        upd = cmin < best
        best = jnp.where(upd, cmin, best)
        bestidx = jnp.where(upd, carg, bestidx)
    codes_ref[0] = bestidx
    quant = jnp.zeros((C, T), dtype=jnp.float32)
    for kc in range(NKCH):
        cb_chunk = cb_ref[kc * KCH:(kc + 1) * KCH, :]
        kio = jax.lax.broadcasted_iota(jnp.int32, (KCH, T), 0) + kc * KCH
        onehot = (kio == bestidx).astype(jnp.float32)
        quant += jax.lax.dot_general(cb_chunk, onehot, (((0,), (0,)), ((), ())),
                                     preferred_element_type=jnp.float32)
    diff = quant - zb
    quant_ref[0] = zb + diff
    b = pl.program_id(0)

    @pl.when(b == 0)
    def _():
        loss_ref[...] = jnp.zeros((1, 1), jnp.float32)

    loss_ref[...] += jnp.sum(diff * diff)[None, None]


def kernel(z, codebook):
    B = z.shape[0]
    codes3, quant, loss_sum = pl.pallas_call(
        _vq_kernel,
        grid=(B,),
        in_specs=[
            pl.BlockSpec((1, C, T), lambda b: (b, 0, 0)),
            pl.BlockSpec((K, C), lambda b: (0, 0)),
        ],
        out_specs=[
            pl.BlockSpec((1, 1, T), lambda b: (b, 0, 0)),
            pl.BlockSpec((1, C, T), lambda b: (b, 0, 0)),
            pl.BlockSpec((1, 1), lambda b: (0, 0)),
        ],
        out_shape=[
            jax.ShapeDtypeStruct((B, 1, T), jnp.int32),
            jax.ShapeDtypeStruct((B, C, T), jnp.float32),
            jax.ShapeDtypeStruct((1, 1), jnp.float32),
        ],
    )(z, codebook)
    codes = codes3.reshape(B, T)
    commit_loss = loss_sum[0, 0] / (B * C * T)
    return codes, quant, commit_loss
